# initial kernel scaffold (unmeasured)
import jax
import jax.numpy as jnp
from jax import lax
from jax.experimental import pallas as pl
from jax.experimental.pallas import tpu as pltpu

N_DEV = 8


def kernel(x, w_mat, scale_x, scale_w):
    if x.dtype != jnp.float8_e5m2:
        x = x.astype(jnp.float8_e5m2)
    if w_mat.dtype != jnp.float8_e5m2:
        w_mat = w_mat.astype(jnp.float8_e5m2)

    k_glob, k_loc = x.shape
    _, n = w_mat.shape
    m_loc = k_glob // N_DEV

    def body(x_ref, w_ref, sx_ref, sw_ref, out_ref, gather_ref, send_sems, recv_sems):
        me = lax.axis_index("i")

        gather_ref[pl.ds(me, 1)] = x_ref[pl.ds(me * m_loc, m_loc), :].reshape(
            1, m_loc, k_loc
        )

        rdmas = []
        for h in range(1, N_DEV):
            t = lax.rem(me + h, N_DEV)
            rdma = pltpu.make_async_remote_copy(
                src_ref=x_ref.at[pl.ds(t * m_loc, m_loc), :],
                dst_ref=gather_ref.at[pl.ds(me, 1)],
                send_sem=send_sems.at[h],
                recv_sem=recv_sems.at[h],
                device_id=(t,),
                device_id_type=pl.DeviceIdType.MESH,
            )
            rdma.start()
            rdmas.append(rdma)

        for rdma in rdmas:
            rdma.wait_recv()

        x_full = jnp.concatenate(
            [gather_ref[k] for k in range(N_DEV)], axis=1
        )
        acc = jnp.dot(x_full, w_ref[:, :], preferred_element_type=jnp.float32)

        y = acc * (sx_ref[0] * sw_ref[0])
        out_ref[:, :] = y * (1.0 / (1.0 + jnp.exp(-jnp.clip(y, -60.0, 60.0))))

        for rdma in rdmas:
            rdma.wait_send()

    return pl.pallas_call(
        body,
        out_shape=jax.ShapeDtypeStruct((m_loc, n), jnp.float32),
        in_specs=[
            pl.BlockSpec(memory_space=pltpu.VMEM),
            pl.BlockSpec(memory_space=pltpu.VMEM),
            pl.BlockSpec(memory_space=pltpu.SMEM),
            pl.BlockSpec(memory_space=pltpu.SMEM),
        ],
        out_specs=pl.BlockSpec(memory_space=pltpu.VMEM),
        scratch_shapes=[
            pltpu.VMEM((N_DEV, m_loc, k_loc), jnp.float8_e5m2),
            pltpu.SemaphoreType.DMA((N_DEV,)),
            pltpu.SemaphoreType.DMA((N_DEV,)),
        ],
        compiler_params=pltpu.CompilerParams(collective_id=0),
    )(x, w_mat, scale_x, scale_w)


# baseline (device time: 46706 ns/iter reference)
import jax
import jax.numpy as jnp
from jax import lax
from jax.experimental import pallas as pl
from jax.experimental.pallas import tpu as pltpu

N_DEV = 8


def kernel(x, w_mat, scale_x, scale_w):
    if x.dtype != jnp.float8_e5m2:
        x = x.astype(jnp.float8_e5m2)
    if w_mat.dtype != jnp.float8_e5m2:
        w_mat = w_mat.astype(jnp.float8_e5m2)

    k_glob, k_loc = x.shape
    _, n = w_mat.shape
    m_loc = k_glob // N_DEV

    def body(x_ref, w_ref, sx_ref, sw_ref, out_ref, gather_ref, send_sems, recv_sems):
        me = lax.axis_index("i")

        gather_ref[pl.ds(me, 1)] = x_ref[pl.ds(me * m_loc, m_loc), :].reshape(
            1, m_loc, k_loc
        )

        rdmas = []
        for h in range(1, N_DEV):
            t = lax.rem(me + h, N_DEV)
            rdma = pltpu.make_async_remote_copy(
                src_ref=x_ref.at[pl.ds(t * m_loc, m_loc), :],
                dst_ref=gather_ref.at[me],
                send_sem=send_sems.at[h],
                recv_sem=recv_sems.at[h],
                device_id=(t,),
                device_id_type=pl.DeviceIdType.MESH,
            )
            rdma.start()
            rdmas.append(rdma)

        for rdma in rdmas:
            rdma.wait_recv()

        x_full = jnp.concatenate(
            [gather_ref[k] for k in range(N_DEV)], axis=1
        )
        acc = jnp.dot(x_full, w_ref[:, :], preferred_element_type=jnp.float32)

        y = acc * (sx_ref[0] * sw_ref[0])
        out_ref[:, :] = y * (1.0 / (1.0 + jnp.exp(-jnp.clip(y, -60.0, 60.0))))

        for rdma in rdmas:
            rdma.wait_send()

    return pl.pallas_call(
        body,
        out_shape=jax.ShapeDtypeStruct((m_loc, n), jnp.float32),
        in_specs=[
            pl.BlockSpec(memory_space=pltpu.VMEM),
            pl.BlockSpec(memory_space=pltpu.VMEM),
            pl.BlockSpec(memory_space=pltpu.SMEM),
            pl.BlockSpec(memory_space=pltpu.SMEM),
        ],
        out_specs=pl.BlockSpec(memory_space=pltpu.VMEM),
        scratch_shapes=[
            pltpu.VMEM((N_DEV, m_loc, k_loc), jnp.float8_e5m2),
            pltpu.SemaphoreType.DMA((N_DEV,)),
            pltpu.SemaphoreType.DMA((N_DEV,)),
        ],
    )(x, w_mat, scale_x, scale_w)


# device time: 37950 ns/iter; 1.2307x vs baseline; 1.2307x over previous
import jax
import jax.numpy as jnp
from jax import lax
from jax.experimental import pallas as pl
from jax.experimental.pallas import tpu as pltpu

N_DEV = 8
W_SLOTS = 4


def kernel(x, w_mat, scale_x, scale_w):
    k_glob, k_loc = x.shape
    _, n = w_mat.shape
    m_loc = k_glob // N_DEV

    def body(x_ref, w_hbm, sx_ref, sw_ref, out_ref,
             xq_ref, gather_ref, wbuf_ref, send_sems, recv_sems, wdma_sems):
        me = lax.axis_index("i")

        xq_ref[:, :] = x_ref[:, :].astype(jnp.float8_e5m2)

        gather_ref[pl.ds(me, 1)] = xq_ref[pl.ds(me * m_loc, m_loc), :].reshape(
            1, m_loc, k_loc
        )

        rdmas = []
        for h in range(1, N_DEV):
            t = lax.rem(me + h, N_DEV)
            rdma = pltpu.make_async_remote_copy(
                src_ref=xq_ref.at[pl.ds(t * m_loc, m_loc), :],
                dst_ref=gather_ref.at[me],
                send_sem=send_sems.at[h],
                recv_sem=recv_sems.at[h],
                device_id=(t,),
                device_id_type=pl.DeviceIdType.MESH,
            )
            rdma.start()
            rdmas.append(rdma)

        def w_dma(h, slot):
            s = lax.rem(me - h + N_DEV, N_DEV)
            return pltpu.make_async_copy(
                w_hbm.at[pl.ds(s * k_loc, k_loc), :],
                wbuf_ref.at[slot],
                wdma_sems.at[slot],
            )

        for h in range(W_SLOTS):
            w_dma(h, h).start()

        acc = None
        for h in range(N_DEV):
            slot = h % W_SLOTS
            w_dma(h, slot).wait()
            if h > 0:
                rdmas[h - 1].wait_recv()
            s = lax.rem(me - h + N_DEV, N_DEV)
            xb = gather_ref[pl.ds(s, 1)].reshape(m_loc, k_loc).astype(jnp.bfloat16)
            wb = wbuf_ref[slot].astype(jnp.bfloat16)
            d = jnp.dot(xb, wb, preferred_element_type=jnp.float32)
            acc = d if acc is None else acc + d
            if h + W_SLOTS < N_DEV:
                w_dma(h + W_SLOTS, slot).start()

        y = acc * (sx_ref[0] * sw_ref[0])
        out_ref[:, :] = y * (1.0 / (1.0 + jnp.exp(-jnp.clip(y, -60.0, 60.0))))

        for rdma in rdmas:
            rdma.wait_send()

    return pl.pallas_call(
        body,
        out_shape=jax.ShapeDtypeStruct((m_loc, n), jnp.float32),
        in_specs=[
            pl.BlockSpec(memory_space=pltpu.VMEM),
            pl.BlockSpec(memory_space=pl.ANY),
            pl.BlockSpec(memory_space=pltpu.SMEM),
            pl.BlockSpec(memory_space=pltpu.SMEM),
        ],
        out_specs=pl.BlockSpec(memory_space=pltpu.VMEM),
        scratch_shapes=[
            pltpu.VMEM((k_glob, k_loc), jnp.float8_e5m2),
            pltpu.VMEM((N_DEV, m_loc, k_loc), jnp.float8_e5m2),
            pltpu.VMEM((W_SLOTS, k_loc, n), jnp.float32),
            pltpu.SemaphoreType.DMA((N_DEV,)),
            pltpu.SemaphoreType.DMA((N_DEV,)),
            pltpu.SemaphoreType.DMA((W_SLOTS,)),
        ],
        compiler_params=pltpu.CompilerParams(
            vmem_limit_bytes=60 * 1024 * 1024,
        ),
    )(x, w_mat, scale_x, scale_w)


# device time: 30051 ns/iter; 1.5542x vs baseline; 1.2629x over previous
import jax
import jax.numpy as jnp
from jax import lax
from jax.experimental import pallas as pl
from jax.experimental.pallas import tpu as pltpu

N_DEV = 8
W_SLOTS = 4
X_SLOTS = 2
MASKS = [1, 3, 2, 4, 5, 7, 6]


def kernel(x, w_mat, scale_x, scale_w):
    k_glob, k_loc = x.shape
    _, n = w_mat.shape
    m_loc = k_glob // N_DEV

    def body(x_hbm, w_hbm, sx_ref, sw_ref, out_ref,
             xbuf_ref, xq_ref, gather_ref, wbuf_ref,
             send_sems, recv_sems, xdma_sems, wdma_sems):
        me = lax.axis_index("i")

        barrier_sem = pltpu.get_barrier_semaphore()
        for m in MASKS:
            pl.semaphore_signal(
                barrier_sem, inc=1,
                device_id=(jnp.bitwise_xor(me, m),),
                device_id_type=pl.DeviceIdType.MESH,
            )

        def x_block_id(c):
            if c < N_DEV - 1:
                return jnp.bitwise_xor(me, MASKS[c])
            return me

        def x_dma(c, slot):
            bid = x_block_id(c)
            return pltpu.make_async_copy(
                x_hbm.at[pl.ds(bid * m_loc, m_loc), :],
                xbuf_ref.at[slot],
                xdma_sems.at[slot],
            )

        for c in range(X_SLOTS):
            x_dma(c, c).start()

        pl.semaphore_wait(barrier_sem, N_DEV - 1)

        send_rdmas = []
        for c in range(N_DEV):
            slot = c % X_SLOTS
            x_dma(c, slot).wait()
            bid = x_block_id(c)
            if c < N_DEV - 1:
                xq_ref[pl.ds(bid * m_loc, m_loc), :] = (
                    xbuf_ref[slot].astype(jnp.float8_e5m2)
                )
                rdma = pltpu.make_async_remote_copy(
                    src_ref=xq_ref.at[pl.ds(bid * m_loc, m_loc), :],
                    dst_ref=gather_ref.at[me],
                    send_sem=send_sems.at[MASKS[c]],
                    recv_sem=recv_sems.at[MASKS[c]],
                    device_id=(bid,),
                    device_id_type=pl.DeviceIdType.MESH,
                )
                rdma.start()
                send_rdmas.append(rdma)
            else:
                gather_ref[pl.ds(me, 1)] = (
                    xbuf_ref[slot].astype(jnp.float8_e5m2).reshape(1, m_loc, k_loc)
                )
            if c + X_SLOTS < N_DEV:
                x_dma(c + X_SLOTS, slot).start()

        def s_of(j):
            if j == 0:
                return me
            return jnp.bitwise_xor(me, MASKS[j - 1])

        def w_dma(j, slot):
            s = s_of(j)
            return pltpu.make_async_copy(
                w_hbm.at[pl.ds(s * k_loc, k_loc), :],
                wbuf_ref.at[slot],
                wdma_sems.at[slot],
            )

        for j in range(W_SLOTS):
            w_dma(j, j).start()

        nh = n // 2
        scale = sx_ref[0] * sw_ref[0]

        def silu(acc_half):
            y = acc_half * scale
            return y * (1.0 / (1.0 + jnp.exp(-jnp.clip(y, -60.0, 60.0))))

        acc0 = acc1 = None
        for j in range(N_DEV):
            slot = j % W_SLOTS
            w_dma(j, slot).wait()
            if j > 0:
                recv = pltpu.make_async_remote_copy(
                    src_ref=xq_ref.at[pl.ds(0, m_loc), :],
                    dst_ref=gather_ref.at[s_of(j)],
                    send_sem=send_sems.at[MASKS[j - 1]],
                    recv_sem=recv_sems.at[MASKS[j - 1]],
                    device_id=(s_of(j),),
                    device_id_type=pl.DeviceIdType.MESH,
                )
                recv.wait_recv()
            s = s_of(j)
            xb = gather_ref[pl.ds(s, 1)].reshape(m_loc, k_loc).astype(jnp.bfloat16)
            wb0 = wbuf_ref[slot, :, :nh].astype(jnp.bfloat16)
            d0 = jnp.dot(xb, wb0, preferred_element_type=jnp.float32)
            acc0 = d0 if acc0 is None else acc0 + d0
            if j == N_DEV - 1:
                out_ref[:, :nh] = silu(acc0)
            wb1 = wbuf_ref[slot, :, nh:].astype(jnp.bfloat16)
            d1 = jnp.dot(xb, wb1, preferred_element_type=jnp.float32)
            acc1 = d1 if acc1 is None else acc1 + d1
            if j + W_SLOTS < N_DEV:
                w_dma(j + W_SLOTS, slot).start()

        out_ref[:, nh:] = silu(acc1)

        for rdma in send_rdmas:
            rdma.wait_send()

    return pl.pallas_call(
        body,
        out_shape=jax.ShapeDtypeStruct((m_loc, n), jnp.float32),
        in_specs=[
            pl.BlockSpec(memory_space=pl.ANY),
            pl.BlockSpec(memory_space=pl.ANY),
            pl.BlockSpec(memory_space=pltpu.SMEM),
            pl.BlockSpec(memory_space=pltpu.SMEM),
        ],
        out_specs=pl.BlockSpec(memory_space=pltpu.VMEM),
        scratch_shapes=[
            pltpu.VMEM((X_SLOTS, m_loc, k_loc), jnp.float32),
            pltpu.VMEM((k_glob, k_loc), jnp.float8_e5m2),
            pltpu.VMEM((N_DEV, m_loc, k_loc), jnp.float8_e5m2),
            pltpu.VMEM((W_SLOTS, k_loc, n), jnp.float32),
            pltpu.SemaphoreType.DMA((N_DEV,)),
            pltpu.SemaphoreType.DMA((N_DEV,)),
            pltpu.SemaphoreType.DMA((X_SLOTS,)),
            pltpu.SemaphoreType.DMA((W_SLOTS,)),
        ],
        compiler_params=pltpu.CompilerParams(
            vmem_limit_bytes=60 * 1024 * 1024,
            collective_id=0,
        ),
    )(x, w_mat, scale_x, scale_w)


# device time: 29239 ns/iter; 1.5974x vs baseline; 1.0278x over previous
import jax
import jax.numpy as jnp
from jax import lax
from jax.experimental import pallas as pl
from jax.experimental.pallas import tpu as pltpu

N_DEV = 8
W_SLOTS = 4
X_SLOTS = 4
MASKS = [1, 3, 2, 4, 5, 7, 6]


def kernel(x, w_mat, scale_x, scale_w):
    k_glob, k_loc = x.shape
    _, n = w_mat.shape
    m_loc = k_glob // N_DEV

    def body(x_hbm, w_hbm, sx_ref, sw_ref, out_ref,
             xbuf_ref, xq_ref, gather_ref, wbuf_ref,
             send_sems, recv_sems, xdma_sems, wdma_sems):
        me = lax.axis_index("i")

        barrier_sem = pltpu.get_barrier_semaphore()
        for m in MASKS:
            pl.semaphore_signal(
                barrier_sem, inc=1,
                device_id=(jnp.bitwise_xor(me, m),),
                device_id_type=pl.DeviceIdType.MESH,
            )

        def s_of(j):
            if j == 0:
                return me
            return jnp.bitwise_xor(me, MASKS[j - 1])

        def w_dma(j, slot):
            s = s_of(j)
            return pltpu.make_async_copy(
                w_hbm.at[pl.ds(s * k_loc, k_loc), :],
                wbuf_ref.at[slot],
                wdma_sems.at[slot],
            )

        def x_block_id(c):
            if c < N_DEV - 1:
                return jnp.bitwise_xor(me, MASKS[c])
            return me

        def x_dma(c, slot):
            bid = x_block_id(c)
            return pltpu.make_async_copy(
                x_hbm.at[pl.ds(bid * m_loc, m_loc), :],
                xbuf_ref.at[slot],
                xdma_sems.at[slot],
            )

        for c in range(X_SLOTS):
            x_dma(c, c).start()

        send_rdmas = []
        for c in range(N_DEV):
            slot = c % X_SLOTS
            x_dma(c, slot).wait()
            bid = x_block_id(c)
            if c < N_DEV - 1:
                xq_ref[pl.ds(bid * m_loc, m_loc), :] = (
                    xbuf_ref[slot].astype(jnp.float8_e5m2)
                )
                if c == 0:
                    pl.semaphore_wait(barrier_sem, N_DEV - 1)
                rdma = pltpu.make_async_remote_copy(
                    src_ref=xq_ref.at[pl.ds(bid * m_loc, m_loc), :],
                    dst_ref=gather_ref.at[me],
                    send_sem=send_sems.at[MASKS[c]],
                    recv_sem=recv_sems.at[MASKS[c]],
                    device_id=(bid,),
                    device_id_type=pl.DeviceIdType.MESH,
                )
                rdma.start()
                send_rdmas.append(rdma)
            else:
                gather_ref[pl.ds(me, 1)] = (
                    xbuf_ref[slot].astype(jnp.float8_e5m2).reshape(1, m_loc, k_loc)
                )
            if c + X_SLOTS < N_DEV:
                x_dma(c + X_SLOTS, slot).start()
            if c >= N_DEV - W_SLOTS:
                w_dma(c - (N_DEV - W_SLOTS), c - (N_DEV - W_SLOTS)).start()

        nh = n // 2
        scale = sx_ref[0] * sw_ref[0]

        def silu(acc_half):
            y = acc_half * scale
            return y * (1.0 / (1.0 + jnp.exp(-jnp.clip(y, -60.0, 60.0))))

        acc0 = acc1 = None
        for j in range(N_DEV):
            slot = j % W_SLOTS
            w_dma(j, slot).wait()
            if j > 0:
                recv = pltpu.make_async_remote_copy(
                    src_ref=xq_ref.at[pl.ds(0, m_loc), :],
                    dst_ref=gather_ref.at[s_of(j)],
                    send_sem=send_sems.at[MASKS[j - 1]],
                    recv_sem=recv_sems.at[MASKS[j - 1]],
                    device_id=(s_of(j),),
                    device_id_type=pl.DeviceIdType.MESH,
                )
                recv.wait_recv()
            s = s_of(j)
            xb = gather_ref[pl.ds(s, 1)].reshape(m_loc, k_loc).astype(jnp.bfloat16)
            wb0 = wbuf_ref[slot, :, :nh].astype(jnp.bfloat16)
            d0 = jnp.dot(xb, wb0, preferred_element_type=jnp.float32)
            acc0 = d0 if acc0 is None else acc0 + d0
            if j == N_DEV - 1:
                out_ref[:, :nh] = silu(acc0)
            wb1 = wbuf_ref[slot, :, nh:].astype(jnp.bfloat16)
            d1 = jnp.dot(xb, wb1, preferred_element_type=jnp.float32)
            acc1 = d1 if acc1 is None else acc1 + d1
            if j + W_SLOTS < N_DEV:
                w_dma(j + W_SLOTS, slot).start()

        out_ref[:, nh:] = silu(acc1)

        for rdma in send_rdmas:
            rdma.wait_send()

    return pl.pallas_call(
        body,
        out_shape=jax.ShapeDtypeStruct((m_loc, n), jnp.float32),
        in_specs=[
            pl.BlockSpec(memory_space=pl.ANY),
            pl.BlockSpec(memory_space=pl.ANY),
            pl.BlockSpec(memory_space=pltpu.SMEM),
            pl.BlockSpec(memory_space=pltpu.SMEM),
        ],
        out_specs=pl.BlockSpec(memory_space=pltpu.VMEM),
        scratch_shapes=[
            pltpu.VMEM((X_SLOTS, m_loc, k_loc), jnp.float32),
            pltpu.VMEM((k_glob, k_loc), jnp.float8_e5m2),
            pltpu.VMEM((N_DEV, m_loc, k_loc), jnp.float8_e5m2),
            pltpu.VMEM((W_SLOTS, k_loc, n), jnp.float32),
            pltpu.SemaphoreType.DMA((N_DEV,)),
            pltpu.SemaphoreType.DMA((N_DEV,)),
            pltpu.SemaphoreType.DMA((X_SLOTS,)),
            pltpu.SemaphoreType.DMA((W_SLOTS,)),
        ],
        compiler_params=pltpu.CompilerParams(
            vmem_limit_bytes=60 * 1024 * 1024,
            collective_id=0,
        ),
    )(x, w_mat, scale_x, scale_w)
